# Initial kernel scaffold; baseline (speedup 1.0000x reference)
import jax
import jax.numpy as jnp
from jax.experimental import pallas as pl


def kernel(pos, edge_index, batch, B, W_edge, b_edge, W_node, b_node, W_cat1, b_cat1, W_cat2, b_cat2, fW1, fb1, g1, be1, fW2, fb2, g2, be2, fW3, fb3, Wg, bg, Wt1, bt1, Wt2, bt2):
    raise NotImplementedError("write your pallas kernel here")


# speedup vs baseline: 3.9035x; 3.9035x over previous
import jax
import jax.numpy as jnp
from jax import lax
from jax.experimental import pallas as pl
from jax.experimental.pallas import tpu as pltpu
from jax.experimental.pallas import tpu_sc as plsc

N = 50000
H = 128
EPS = 1e-5

_NC, _NS = 2, 16
_Q = 32
_GC = 128
_EPT = 50176
_NCH = _EPT // _GC
_ACC = 50048
_ZSPAN = _ACC // _NS


def _seg_body(src_hbm, dst_hbm, wr_hbm, out_hbm,
              acc, sbuf0, sbuf1, dbuf, gidx0, gidx1, rows0, rows1, zbuf,
              sem0, sem1, semS0, semS1):
    c = lax.axis_index("c")
    s = lax.axis_index("s")
    ebase = s * _EPT

    zv = jnp.zeros((16,), jnp.float32)

    def _zb(t, carry):
        zbuf[t // 2, pl.ds((t % 2) * 16, 16)] = zv
        return carry

    lax.fori_loop(0, 256, _zb, 0)

    def _load_async(j, sbuf, slot, sem):
        pltpu.async_copy(src_hbm.at[pl.ds(ebase + j * _GC, _GC)], sbuf, sem)
        pltpu.async_copy(dst_hbm.at[pl.ds(ebase + j * _GC, _GC)],
                         dbuf.at[slot], sem)

    def _wait_load(j, sbuf, slot, sem):
        pltpu.make_async_copy(src_hbm.at[pl.ds(ebase + j * _GC, _GC)],
                              sbuf, sem).wait()
        pltpu.make_async_copy(dst_hbm.at[pl.ds(ebase + j * _GC, _GC)],
                              dbuf.at[slot], sem).wait()

    def _fire(gidx, sbuf, rows, sem, q):
        for t in range(8):
            sv = sbuf[pl.ds(t * 16, 16)]
            gidx[pl.ds(t * 16, 16)] = sv * 4 + q
        pltpu.async_copy(wr_hbm.at[gidx], rows, sem)

    def _wait_gather(gidx, rows, sem):
        pltpu.make_async_copy(wr_hbm.at[gidx], rows, sem).wait()

    for k in range(2):
        q = 2 * k + c
        obase = q * N

        zb = s * _ZSPAN
        for t in range(24):
            pltpu.sync_copy(zbuf, acc.at[pl.ds(zb + t * 128, 128)])
        pltpu.sync_copy(zbuf.at[pl.ds(0, 56)], acc.at[pl.ds(zb + 3072, 56)])
        plsc.subcore_barrier()

        pltpu.sync_copy(src_hbm.at[pl.ds(ebase, _GC)], sbuf0)
        pltpu.sync_copy(dst_hbm.at[pl.ds(ebase, _GC)], dbuf.at[0])
        _fire(gidx0, sbuf0, rows0, sem0, q)
        _load_async(1, sbuf1, 1, semS1)

        def _pair(jj, carry):
            j0 = 2 * jj
            _wait_load(j0 + 1, sbuf1, 1, semS1)
            _fire(gidx1, sbuf1, rows1, sem1, q)
            _wait_gather(gidx0, rows0, sem0)
            pltpu.sync_copy(rows0, acc.at[dbuf.at[0]], add=True)

            @pl.when(jj < _NCH // 2 - 1)
            def _():
                _load_async(j0 + 2, sbuf0, 0, semS0)

            _wait_gather(gidx1, rows1, sem1)
            pltpu.sync_copy(rows1, acc.at[dbuf.at[1]], add=True)

            @pl.when(jj < _NCH // 2 - 1)
            def _():
                _wait_load(j0 + 2, sbuf0, 0, semS0)
                _fire(gidx0, sbuf0, rows0, sem0, q)
                _load_async(j0 + 3, sbuf1, 1, semS1)

            return carry

        lax.fori_loop(0, _NCH // 2, _pair, 0)
        plsc.subcore_barrier()

        @pl.when(s < _NS - 1)
        def _():
            pltpu.sync_copy(acc.at[pl.ds(s * _ZSPAN, _ZSPAN)],
                            out_hbm.at[pl.ds(obase + s * _ZSPAN, _ZSPAN)])

        @pl.when(s == _NS - 1)
        def _():
            last = N - (_NS - 1) * _ZSPAN
            pltpu.sync_copy(acc.at[pl.ds((_NS - 1) * _ZSPAN, last)],
                            out_hbm.at[pl.ds(obase + (_NS - 1) * _ZSPAN, last)])

        plsc.subcore_barrier()


def _segment_sum_sc(src_pad, dst_pad, w_r):
    mesh = plsc.VectorSubcoreMesh(
        core_axis_name="c", subcore_axis_name="s",
        num_cores=_NC, num_subcores=_NS)
    f = pl.kernel(
        _seg_body,
        out_type=jax.ShapeDtypeStruct((4 * N, _Q), jnp.float32),
        mesh=mesh,
        compiler_params=pltpu.CompilerParams(use_tc_tiling_on_sc=False),
        scratch_types=[
            pltpu.VMEM_SHARED((_ACC, _Q), jnp.float32),
            pltpu.VMEM((_GC,), jnp.int32),
            pltpu.VMEM((_GC,), jnp.int32),
            pltpu.VMEM((2, _GC), jnp.int32),
            pltpu.VMEM((_GC,), jnp.int32),
            pltpu.VMEM((_GC,), jnp.int32),
            pltpu.VMEM((_GC, _Q), jnp.float32),
            pltpu.VMEM((_GC, _Q), jnp.float32),
            pltpu.VMEM((128, _Q), jnp.float32),
            pltpu.SemaphoreType.DMA,
            pltpu.SemaphoreType.DMA,
            pltpu.SemaphoreType.DMA,
            pltpu.SemaphoreType.DMA,
        ],
    )
    return f(src_pad, dst_pad, w_r)


_BLK = 2000
_GRID = N // _BLK


def _bdot(a, b):
    return jnp.dot(a.astype(jnp.bfloat16), b.astype(jnp.bfloat16),
                   preferred_element_type=jnp.float32)


def _stats_update(h, s_ref, q_ref):
    bs = jnp.sum(h, axis=0, keepdims=True)
    bm = bs * (1.0 / _BLK)
    d = h - bm
    m2b = jnp.sum(d * d, axis=0, keepdims=True)
    i = pl.program_id(0)

    @pl.when(i == 0)
    def _():
        s_ref[...] = bs
        q_ref[...] = m2b

    @pl.when(i > 0)
    def _():
        no = (i * _BLK).astype(jnp.float32)
        delta = bm - s_ref[...] * (1.0 / no)
        q_ref[...] = (q_ref[...] + m2b
                      + delta * delta * (no * _BLK / (no + _BLK)))
        s_ref[...] = s_ref[...] + bs


def _k1_body(pos_ref, seg4_ref, Bt_ref, Wnt_ref, bn_ref, M2_ref, c1_ref,
             M1_ref, be_ref, W1_ref, b1_ref, h1_ref, s_ref, q_ref):
    p = pos_ref[...]
    vp = 2.0 * jnp.pi * _bdot(p, Bt_ref[...])
    x = jnp.concatenate([jnp.cos(vp), jnp.sin(vp)], axis=-1)
    seg4 = seg4_ref[...]
    seg = jnp.concatenate([seg4[0], seg4[1], seg4[2], seg4[3]], axis=-1)
    out0 = seg + be_ref[...]
    out = out0 + _bdot(out0, M1_ref[...]) + c1_ref[...]
    xn = _bdot(x, Wnt_ref[...]) + bn_ref[...]
    out = out + xn + _bdot(xn, M2_ref[...])
    h = jnp.maximum(out, 0.0)
    h1 = jnp.maximum(_bdot(h, W1_ref[...]) + b1_ref[...], 0.0)
    h1_ref[...] = h1
    _stats_update(h1, s_ref, q_ref)


def _k2_body(h1_ref, s_ref, q_ref, g_ref, be_ref, W2_ref, b2_ref,
             h2_ref, s2_ref, q2_ref):
    mean = s_ref[...] * (1.0 / N)
    var = q_ref[...] * (1.0 / N)
    scale = g_ref[...] * lax.rsqrt(var + EPS)
    shift = be_ref[...] - mean * scale
    h1n = h1_ref[...] * scale + shift
    h2 = jnp.maximum(_bdot(h1n, W2_ref[...]) + b2_ref[...], 0.0)
    h2_ref[...] = h2
    _stats_update(h2, s2_ref, q2_ref)


def _k3_body(h2_ref, s_ref, q_ref, g_ref, be_ref, W3_ref, b3_ref,
             x2_ref, mx_ref):
    mean = s_ref[...] * (1.0 / N)
    var = q_ref[...] * (1.0 / N)
    scale = g_ref[...] * lax.rsqrt(var + EPS)
    shift = be_ref[...] - mean * scale
    h2n = h2_ref[...] * scale + shift
    x2 = _bdot(h2n, W3_ref[...]) + b3_ref[...]
    x2_ref[...] = x2

    @pl.when(pl.program_id(0) == 0)
    def _():
        mx_ref[...] = jnp.full_like(mx_ref, -jnp.inf)

    mx_ref[...] = jnp.maximum(mx_ref[...], jnp.max(x2, axis=0, keepdims=True))


def _k4_body(x2_ref, mx_ref, Wg_ref, bg_ref, W1a_ref, W1b_ref, bt1_ref,
             W2t_ref, bt2_ref, t_ref, xcat_ref):
    hmax = mx_ref[...]
    hg = _bdot(hmax, Wg_ref[...]) + bg_ref[...]
    hg = jnp.maximum(hg, 0.2 * hg)
    x2 = x2_ref[...]
    u = (_bdot(x2, W1a_ref[...]) + _bdot(hg, W1b_ref[...]) + bt1_ref[...])
    u = jnp.maximum(u, 0.2 * u)
    t = jnp.tanh(_bdot(u, W2t_ref[...]) + bt2_ref[...])
    t_ref[...] = t
    xcat_ref[...] = jnp.concatenate(
        [x2, jnp.broadcast_to(hg, x2.shape)], axis=-1)


def _row_spec(width):
    return pl.BlockSpec((_BLK, width), lambda i: (i, 0))


def _full_spec(shape):
    return pl.BlockSpec(shape, lambda i: tuple(0 for _ in shape))


def kernel(pos, edge_index, batch, B, W_edge, b_edge, W_node, b_node,
           W_cat1, b_cat1, W_cat2, b_cat2, fW1, fb1, g1, be1, fW2, fb2,
           g2, be2, fW3, fb3, Wg, bg, Wt1, bt1, Wt2, bt2):
    f32 = jnp.float32
    M1 = W_cat1.T
    M2 = W_cat2.T
    c1 = (b_cat1 + b_cat2).reshape(1, H)

    E = edge_index.shape[1]
    npad = _NS * _EPT - E
    src_pad = jnp.concatenate(
        [edge_index[0], jnp.zeros((npad,), jnp.int32)])
    dst_pad = jnp.concatenate(
        [edge_index[1], jnp.full((npad,), N, jnp.int32)])
    w_r = W_edge.reshape(-1, _Q)

    seg_flat = _segment_sum_sc(src_pad, dst_pad, w_r)
    seg4 = seg_flat.reshape(4, N, _Q)

    row128 = _row_spec(H)
    w128 = _full_spec((H, H))
    vec128 = _full_spec((1, H))
    seg4_spec = pl.BlockSpec((4, _BLK, _Q), lambda i: (0, i, 0))

    pos8 = jnp.zeros((N, 8), f32).at[:, :3].set(pos)
    Bt8 = jnp.zeros((8, 64), f32).at[:3, :].set(B.T)

    h1, s1, q1 = pl.pallas_call(
        _k1_body,
        grid=(_GRID,),
        in_specs=[_row_spec(8), seg4_spec, _full_spec((8, 64)), w128, vec128,
                  w128, vec128, w128, vec128, w128, vec128],
        out_specs=[row128, vec128, vec128],
        out_shape=[jax.ShapeDtypeStruct((N, H), f32),
                   jax.ShapeDtypeStruct((1, H), f32),
                   jax.ShapeDtypeStruct((1, H), f32)],
    )(pos8, seg4, Bt8, W_node.T, b_node.reshape(1, H), M2, c1,
      M1, b_edge.reshape(1, H), fW1.T, fb1.reshape(1, H))

    h2, s2, q2 = pl.pallas_call(
        _k2_body,
        grid=(_GRID,),
        in_specs=[row128, vec128, vec128, vec128, vec128, w128, vec128],
        out_specs=[row128, vec128, vec128],
        out_shape=[jax.ShapeDtypeStruct((N, H), f32),
                   jax.ShapeDtypeStruct((1, H), f32),
                   jax.ShapeDtypeStruct((1, H), f32)],
    )(h1, s1, q1, g1.reshape(1, H), be1.reshape(1, H), fW2.T,
      fb2.reshape(1, H))

    x2, mx = pl.pallas_call(
        _k3_body,
        grid=(_GRID,),
        in_specs=[row128, vec128, vec128, vec128, vec128, w128, vec128],
        out_specs=[row128, vec128],
        out_shape=[jax.ShapeDtypeStruct((N, H), f32),
                   jax.ShapeDtypeStruct((1, H), f32)],
    )(h2, s2, q2, g2.reshape(1, H), be2.reshape(1, H), fW3.T,
      fb3.reshape(1, H))

    Wt2p = jnp.zeros((H, 8), f32).at[:, :3].set(Wt2.T)
    bt2p = jnp.zeros((1, 8), f32).at[:, :3].set(bt2.reshape(1, 3))

    t8, xcat = pl.pallas_call(
        _k4_body,
        grid=(_GRID,),
        in_specs=[row128, vec128, w128, vec128, w128, w128, vec128,
                  _full_spec((H, 8)), _full_spec((1, 8))],
        out_specs=[_row_spec(8), _row_spec(2 * H)],
        out_shape=[jax.ShapeDtypeStruct((N, 8), f32),
                   jax.ShapeDtypeStruct((N, 2 * H), f32)],
    )(x2, mx, Wg.T, bg.reshape(1, H), Wt1.T[:H], Wt1.T[H:],
      bt1.reshape(1, H), Wt2p, bt2p)

    return (t8[:, :3], xcat)
